# baseline (speedup 1.0000x reference)
import functools

import jax
import jax.numpy as jnp
from jax import lax
from jax.experimental import pallas as pl
from jax.experimental.pallas import tpu as pltpu
from jax.experimental.pallas import tpu_sc as plsc

N = 16384
K = 1024
D = 256
TN = 4096
GRID = N // TN
COMMIT = 0.25

NW = 32
ROWS_PER_W = (N // 2) // NW
CHUNK = 128
NCHUNK = ROWS_PER_W // CHUNK


HALF = N // 2
GRID2 = HALF // TN


def _dist_body(z_ref, w_ref, idx_ref, loss_ref, wsq_ref):
    i = pl.program_id(0)

    @pl.when(i == 0)
    def _():
        wsq_ref[...] = jnp.sum(w_ref[...] ** 2, axis=1).reshape(1, K)

    s = lax.dot_general(z_ref[...], w_ref[...],
                        (((1,), (1,)), ((), ())),
                        preferred_element_type=jnp.float32)
    zsq = jnp.sum(z_ref[...] ** 2, axis=1, keepdims=True)
    d = (zsq + wsq_ref[...]) - 2.0 * s
    m = jnp.min(d, axis=1, keepdims=True)
    iota = lax.broadcasted_iota(jnp.int32, (TN, K), 1).astype(jnp.float32)
    idxf = jnp.min(jnp.where(d == m, iota, float(K)), axis=1, keepdims=True)
    idx_ref[...] = idxf.astype(jnp.int32)

    @pl.when(i == 0)
    def _():
        loss_ref[0, 0] = 0.0

    loss_ref[0, 0] += jnp.sum(m)

    @pl.when(i == GRID2 - 1)
    def _():
        loss_ref[0, 0] = loss_ref[0, 0] * ((1.0 + COMMIT) / (N * D))


def _dist_half(z, W, h):
    return pl.pallas_call(
        _dist_body,
        grid=(GRID2,),
        in_specs=[
            pl.BlockSpec((TN, D), lambda i, _h=h: (i + _h * GRID2, 0)),
            pl.BlockSpec((K, D), lambda i: (0, 0)),
        ],
        scratch_shapes=[pltpu.VMEM((1, K), jnp.float32)],
        out_specs=[
            pl.BlockSpec((TN, 1), lambda i: (i, 0)),
            pl.BlockSpec((1, 1), lambda i: (0, 0), memory_space=pltpu.SMEM),
        ],
        out_shape=[
            jax.ShapeDtypeStruct((HALF, 1), jnp.int32),
            jax.ShapeDtypeStruct((1, 1), jnp.float32),
        ],
    )(z, W)


@functools.cache
def _make_gather():
    @functools.partial(
        pl.kernel,
        mesh=plsc.VectorSubcoreMesh(core_axis_name="c", subcore_axis_name="s"),
        out_type=jax.ShapeDtypeStruct((N // 2, D), jnp.float32),
        scratch_types=[
            pltpu.VMEM((ROWS_PER_W,), jnp.int32),
            pltpu.VMEM((CHUNK, D), jnp.float32),
            pltpu.VMEM((CHUNK, D), jnp.float32),
            pltpu.VMEM((CHUNK, D), jnp.float32),
            pltpu.SemaphoreType.DMA,
            pltpu.SemaphoreType.DMA,
            pltpu.SemaphoreType.DMA,
            pltpu.SemaphoreType.DMA,
            pltpu.SemaphoreType.DMA,
            pltpu.SemaphoreType.DMA,
        ],
    )
    def _gather(w_hbm, idx_hbm, out_hbm, idx_all,
                buf0, buf1, buf2, g0, g1, g2, w0, w1, w2):
        wid = lax.axis_index("s") * 2 + lax.axis_index("c")
        base0 = wid * ROWS_PER_W
        pltpu.sync_copy(idx_hbm.at[pl.ds(base0, ROWS_PER_W)], idx_all)
        bufs, gs, ws = (buf0, buf1, buf2), (g0, g1, g2), (w0, w1, w2)
        nb = 3
        hg, hw = [None] * NCHUNK, [None] * NCHUNK
        for c in range(NCHUNK):
            b = c % nb
            if c >= nb:
                hw[c - nb].wait()
            hg[c] = pltpu.async_copy(
                w_hbm.at[idx_all.at[pl.ds(c * CHUNK, CHUNK)]], bufs[b], gs[b])
            if c >= 1:
                pb = (c - 1) % nb
                hg[c - 1].wait()
                hw[c - 1] = pltpu.async_copy(
                    bufs[pb], out_hbm.at[pl.ds(base0 + (c - 1) * CHUNK, CHUNK)],
                    ws[pb])
        last = NCHUNK - 1
        hg[last].wait()
        hw[last] = pltpu.async_copy(
            bufs[last % nb], out_hbm.at[pl.ds(base0 + last * CHUNK, CHUNK)],
            ws[last % nb])
        for c in range(max(0, NCHUNK - nb), NCHUNK):
            hw[c].wait()

    return _gather


def kernel(z, W):
    g = _make_gather()
    ia, la = _dist_half(z, W, 0)
    qa = g(W, ia.reshape(HALF))
    ib, lb = _dist_half(z, W, 1)
    qb = g(W, ib.reshape(HALF))
    z_q = jnp.concatenate([qa, qb], axis=0)
    idx = jnp.concatenate([ia.reshape(HALF), ib.reshape(HALF)])
    loss = la[0, 0] + lb[0, 0]
    return (z_q, loss, idx)


# speedup vs baseline: 1.2007x; 1.2007x over previous
import functools

import jax
import jax.numpy as jnp
from jax import lax
from jax.experimental import pallas as pl
from jax.experimental.pallas import tpu as pltpu
from jax.experimental.pallas import tpu_sc as plsc

N = 16384
K = 1024
D = 256
TN = 4096
GRID = N // TN
COMMIT = 0.25

NW = 32
ROWS_PER_W = N // NW
CHUNK = 128
NCHUNK = ROWS_PER_W // CHUNK


def _dist_body(z_ref, w_ref, idx_ref, loss_ref, wsq_ref):
    i = pl.program_id(0)

    @pl.when(i == 0)
    def _():
        wsq_ref[...] = jnp.sum(w_ref[...] ** 2, axis=1).reshape(1, K)

    s = lax.dot_general(z_ref[...], w_ref[...],
                        (((1,), (1,)), ((), ())),
                        preferred_element_type=jnp.float32)
    zsq = jnp.sum(z_ref[...] ** 2, axis=1, keepdims=True)
    d = (zsq + wsq_ref[...]) - 2.0 * s
    m = jnp.min(d, axis=1, keepdims=True)
    iota = lax.broadcasted_iota(jnp.int32, (TN, K), 1).astype(jnp.float32)
    idxf = jnp.min(jnp.where(d == m, iota, float(K)), axis=1, keepdims=True)
    idx_ref[...] = idxf.astype(jnp.int32)

    @pl.when(i == 0)
    def _():
        loss_ref[0, 0] = 0.0

    loss_ref[0, 0] += jnp.sum(m)

    @pl.when(i == GRID - 1)
    def _():
        loss_ref[0, 0] = loss_ref[0, 0] * ((1.0 + COMMIT) / (N * D))


def _dist(z, W):
    return pl.pallas_call(
        _dist_body,
        grid=(GRID,),
        in_specs=[
            pl.BlockSpec((TN, D), lambda i: (i, 0)),
            pl.BlockSpec((K, D), lambda i: (0, 0)),
        ],
        scratch_shapes=[pltpu.VMEM((1, K), jnp.float32)],
        out_specs=[
            pl.BlockSpec((TN, 1), lambda i: (i, 0)),
            pl.BlockSpec((1, 1), lambda i: (0, 0), memory_space=pltpu.SMEM),
        ],
        out_shape=[
            jax.ShapeDtypeStruct((N, 1), jnp.int32),
            jax.ShapeDtypeStruct((1, 1), jnp.float32),
        ],
    )(z, W)


@functools.cache
def _make_gather():
    @functools.partial(
        pl.kernel,
        mesh=plsc.VectorSubcoreMesh(core_axis_name="c", subcore_axis_name="s"),
        out_type=jax.ShapeDtypeStruct((N, D), jnp.float32),
        scratch_types=[
            pltpu.VMEM((ROWS_PER_W,), jnp.int32),
            pltpu.VMEM((CHUNK, D), jnp.float32),
            pltpu.VMEM((CHUNK, D), jnp.float32),
            pltpu.VMEM((CHUNK, D), jnp.float32),
            pltpu.SemaphoreType.DMA,
            pltpu.SemaphoreType.DMA,
            pltpu.SemaphoreType.DMA,
            pltpu.SemaphoreType.DMA,
            pltpu.SemaphoreType.DMA,
            pltpu.SemaphoreType.DMA,
        ],
    )
    def _gather(w_hbm, idx_hbm, out_hbm, idx_all,
                buf0, buf1, buf2, g0, g1, g2, w0, w1, w2):
        wid = lax.axis_index("s") * 2 + lax.axis_index("c")
        base0 = wid * ROWS_PER_W
        pltpu.sync_copy(idx_hbm.at[pl.ds(base0, ROWS_PER_W)], idx_all)
        bufs, gs, ws = (buf0, buf1, buf2), (g0, g1, g2), (w0, w1, w2)
        nb = 3
        hg, hw = [None] * NCHUNK, [None] * NCHUNK
        for c in range(NCHUNK):
            b = c % nb
            if c >= nb:
                hw[c - nb].wait()
            hg[c] = pltpu.async_copy(
                w_hbm.at[idx_all.at[pl.ds(c * CHUNK, CHUNK)]], bufs[b], gs[b])
            if c >= 1:
                pb = (c - 1) % nb
                hg[c - 1].wait()
                hw[c - 1] = pltpu.async_copy(
                    bufs[pb], out_hbm.at[pl.ds(base0 + (c - 1) * CHUNK, CHUNK)],
                    ws[pb])
        last = NCHUNK - 1
        hg[last].wait()
        hw[last] = pltpu.async_copy(
            bufs[last % nb], out_hbm.at[pl.ds(base0 + last * CHUNK, CHUNK)],
            ws[last % nb])
        for c in range(max(0, NCHUNK - nb), NCHUNK):
            hw[c].wait()

    return _gather


def kernel(z, W):
    idx2, loss = _dist(z, W)
    idx = idx2.reshape(N)
    z_q = _make_gather()(W, idx)
    return (z_q, loss[0, 0], idx)
